# baseline (speedup 1.0000x reference)
import functools

import jax
import jax.numpy as jnp
from jax import lax
from jax.experimental import pallas as pl
from jax.experimental.pallas import tpu as pltpu
from jax.experimental.pallas import tpu_sc as plsc

EMB = 64
HID = 128
TAGS = 50
TPAD = 64

NC = 2
NS = 16
NW = NC * NS
CHUNK = 128

VBLK = 4096
QUART = 1 << 18


def _fold_body(w1_ref, w2_ref, b1_ref, b2_ref, wc_ref, bc_ref):
    w2 = w2_ref[...]
    pad = jnp.zeros((HID, TPAD - TAGS), jnp.float32)
    w2p = jnp.concatenate([w2, pad], axis=1)
    wc_ref[...] = jnp.dot(w1_ref[...], w2p, preferred_element_type=jnp.float32)
    bc_ref[...] = jnp.dot(b1_ref[...], w2p, preferred_element_type=jnp.float32) + \
        jnp.concatenate([b2_ref[...], jnp.zeros((1, TPAD - TAGS), jnp.float32)], axis=1)


def _fold(W1, b1, W2, b2):
    return pl.pallas_call(
        _fold_body,
        out_shape=(jax.ShapeDtypeStruct((EMB, TPAD), jnp.float32),
                   jax.ShapeDtypeStruct((1, TPAD), jnp.float32)),
    )(W1, W2, b1.reshape(1, HID), b2.reshape(1, TAGS))


def _vocab_body(e0_ref, e1_ref, e2_ref, e3_ref, wc_ref, bc_ref, o_ref):
    wc = wc_ref[...].astype(jnp.bfloat16)
    bc = bc_ref[...]

    def tags_of(e_ref):
        e = e_ref[...].astype(jnp.bfloat16)
        return jax.lax.dot_general(e, wc, (((0,), (0,)), ((), ())),
                                   preferred_element_type=jnp.float32) + bc

    m0, m1, m2, m3 = (tags_of(r) for r in (e0_ref, e1_ref, e2_ref, e3_ref))
    lo = jnp.concatenate([m0, m1], axis=1)
    hi = jnp.concatenate([m2, m3], axis=1)
    lo_u = lax.bitcast_convert_type(lo.astype(jnp.bfloat16).astype(jnp.float32),
                                    jnp.uint32)
    hi_u = lax.bitcast_convert_type(hi.astype(jnp.bfloat16).astype(jnp.float32),
                                    jnp.uint32)
    packed = (lo_u >> 16) | (hi_u & jnp.uint32(0xFFFF0000))
    o_ref[...] = lax.bitcast_convert_type(packed, jnp.float32)


def _vocab_tags(embT, wc, bc):
    v = embT.shape[1]
    last_blk = pl.cdiv(v, VBLK) - 1
    off = QUART // VBLK

    def mk(j):
        return pl.BlockSpec(
            (EMB, VBLK), lambda i, j=j: (0, jnp.minimum(i + j * off, last_blk)))

    return pl.pallas_call(
        _vocab_body,
        grid=(off,),
        in_specs=[
            mk(0), mk(1), mk(2), mk(3),
            pl.BlockSpec((EMB, TPAD), lambda i: (0, 0)),
            pl.BlockSpec((1, TPAD), lambda i: (0, 0)),
        ],
        out_specs=pl.BlockSpec((VBLK, 2 * TPAD), lambda i: (i, 0)),
        out_shape=jax.ShapeDtypeStruct((QUART, 2 * TPAD), jnp.float32),
        compiler_params=pltpu.CompilerParams(fuse_transposed_lhs_in_matmul=True),
    )(embT, embT, embT, embT, wc, bc)


QLANE = 50


def _sc_gather(table, sent):
    n = sent.shape[0]
    b_per_w = n // NW
    nchunk = b_per_w // CHUNK
    nvec = CHUNK // 16
    mesh = plsc.VectorSubcoreMesh(core_axis_name="c", subcore_axis_name="s")

    @functools.partial(
        pl.kernel,
        out_type=jax.ShapeDtypeStruct((n, 2 * TPAD), jnp.float32),
        mesh=mesh,
        scratch_types=[
            pltpu.VMEM((nchunk, CHUNK), jnp.int32),
            pltpu.VMEM((nchunk, CHUNK), jnp.float32),
            pltpu.VMEM((nchunk, CHUNK, 2 * TPAD), jnp.float32),
            pltpu.SemaphoreType.DMA,
        ],
        compiler_params=pltpu.CompilerParams(use_tc_tiling_on_sc=True, needs_layout_passes=False),
    )
    def k(table_hbm, sent_hbm, out_hbm, idx_v, q_v, rows_v, sem):
        wid = lax.axis_index("s") * NC + lax.axis_index("c")
        base = wid * b_per_w
        for j in range(nchunk):
            pltpu.sync_copy(sent_hbm.at[pl.ds(base + j * CHUNK, CHUNK)],
                            idx_v.at[j])
        for j in range(nchunk):
            for v in range(nvec):
                sl = pl.ds(v * 16, 16)
                s_vec = idx_v[j, sl]
                q_v[j, sl] = plsc.bitcast(s_vec >> 18, jnp.float32)
                idx_v[j, sl] = s_vec & (QUART - 1)
        copies = [
            pltpu.async_copy(table_hbm.at[idx_v.at[j]], rows_v.at[j], sem)
            for j in range(nchunk)
        ]
        lanes = jnp.full((16,), QLANE, jnp.int32)
        for j in range(nchunk):
            copies[j].wait()
            jj = jnp.full((16,), j, jnp.int32)
            for v in range(nvec):
                rowv = lax.iota(jnp.int32, 16) + (v * 16)
                plsc.store_scatter(rows_v, [jj, rowv, lanes],
                                   q_v[j, pl.ds(v * 16, 16)])
            pltpu.sync_copy(rows_v.at[j],
                            out_hbm.at[pl.ds(base + j * CHUNK, CHUNK)])

    return k(table, sent)


def _final_body(e_ref, o_ref):
    u = lax.bitcast_convert_type(e_ref[...], jnp.uint32)
    q = u[:, QLANE:QLANE + 1]
    sel = jnp.where((q & 2) == 0, u << 16, u & jnp.uint32(0xFFFF0000))
    t128 = lax.bitcast_convert_type(sel, jnp.float32)
    t = jnp.where((q & 1) == 0, t128[:, :TPAD], t128[:, TPAD:])
    t = t[:, :TAGS]
    x = t - jnp.max(t, axis=1, keepdims=True)
    o_ref[...] = x - jnp.log(jnp.sum(jnp.exp(x), axis=1, keepdims=True))


def _final(rows, interpret=False):
    n = rows.shape[0]
    blk = min(n, 8192)
    return pl.pallas_call(
        _final_body,
        grid=(n // blk,),
        in_specs=[
            pl.BlockSpec((blk, 2 * TPAD), lambda i: (i, 0)),
        ],
        out_specs=pl.BlockSpec((blk, TAGS), lambda i: (i, 0)),
        out_shape=jax.ShapeDtypeStruct((n, TAGS), jnp.float32),
        interpret=interpret,
    )(rows)


def kernel(sentence, emb, W1, b1, W2, b2):
    s32 = sentence.astype(jnp.int32)
    wc, bc = _fold(W1, b1, W2, b2)
    pq = _vocab_tags(emb.T, wc, bc)
    rows = _sc_gather(pq, s32)
    return _final(rows)


# speedup vs baseline: 1.0870x; 1.0870x over previous
import functools

import jax
import jax.numpy as jnp
from jax import lax
from jax.experimental import pallas as pl
from jax.experimental.pallas import tpu as pltpu
from jax.experimental.pallas import tpu_sc as plsc

EMB = 64
HID = 128
TAGS = 50
TPAD = 64

NC = 2
NS = 16
NW = NC * NS
CHUNK = 128

VBLK = 8192
QUART = 1 << 18


def _fold_body(w1_ref, w2_ref, b1_ref, b2_ref, wc_ref, bc_ref):
    w2 = w2_ref[...]
    pad = jnp.zeros((HID, TPAD - TAGS), jnp.float32)
    w2p = jnp.concatenate([w2, pad], axis=1)
    wc_ref[...] = jnp.dot(w1_ref[...], w2p, preferred_element_type=jnp.float32)
    bc_ref[...] = jnp.dot(b1_ref[...], w2p, preferred_element_type=jnp.float32) + \
        jnp.concatenate([b2_ref[...], jnp.zeros((1, TPAD - TAGS), jnp.float32)], axis=1)


def _fold(W1, b1, W2, b2):
    return pl.pallas_call(
        _fold_body,
        out_shape=(jax.ShapeDtypeStruct((EMB, TPAD), jnp.float32),
                   jax.ShapeDtypeStruct((1, TPAD), jnp.float32)),
    )(W1, W2, b1.reshape(1, HID), b2.reshape(1, TAGS))


def _vocab_body(e0_ref, e1_ref, e2_ref, e3_ref, wc_ref, bc_ref, o_ref):
    wc = wc_ref[...].astype(jnp.bfloat16)
    bc = bc_ref[...]

    def tags_of(e_ref):
        e = e_ref[...].astype(jnp.bfloat16)
        return jax.lax.dot_general(e, wc, (((0,), (0,)), ((), ())),
                                   preferred_element_type=jnp.float32) + bc

    m0, m1, m2, m3 = (tags_of(r) for r in (e0_ref, e1_ref, e2_ref, e3_ref))
    lo = jnp.concatenate([m0, m1], axis=1)
    hi = jnp.concatenate([m2, m3], axis=1)
    lo_u = lax.bitcast_convert_type(lo.astype(jnp.bfloat16).astype(jnp.float32),
                                    jnp.uint32)
    hi_u = lax.bitcast_convert_type(hi.astype(jnp.bfloat16).astype(jnp.float32),
                                    jnp.uint32)
    packed = (lo_u >> 16) | (hi_u & jnp.uint32(0xFFFF0000))
    o_ref[...] = lax.bitcast_convert_type(packed, jnp.float32)


def _vocab_tags(embT, wc, bc):
    v = embT.shape[1]
    last_blk = pl.cdiv(v, VBLK) - 1
    off = QUART // VBLK

    def mk(j):
        return pl.BlockSpec(
            (EMB, VBLK), lambda i, j=j: (0, jnp.minimum(i + j * off, last_blk)))

    return pl.pallas_call(
        _vocab_body,
        grid=(off,),
        in_specs=[
            mk(0), mk(1), mk(2), mk(3),
            pl.BlockSpec((EMB, TPAD), lambda i: (0, 0)),
            pl.BlockSpec((1, TPAD), lambda i: (0, 0)),
        ],
        out_specs=pl.BlockSpec((VBLK, 2 * TPAD), lambda i: (i, 0)),
        out_shape=jax.ShapeDtypeStruct((QUART, 2 * TPAD), jnp.float32),
        compiler_params=pltpu.CompilerParams(fuse_transposed_lhs_in_matmul=True),
    )(embT, embT, embT, embT, wc, bc)


QLANE = 50


def _sc_gather(table, sent):
    n = sent.shape[0]
    b_per_w = n // NW
    nchunk = b_per_w // CHUNK
    nvec = CHUNK // 16
    mesh = plsc.VectorSubcoreMesh(core_axis_name="c", subcore_axis_name="s")

    @functools.partial(
        pl.kernel,
        out_type=jax.ShapeDtypeStruct((n, 2 * TPAD), jnp.float32),
        mesh=mesh,
        scratch_types=[
            pltpu.VMEM((nchunk, CHUNK), jnp.int32),
            pltpu.VMEM((nchunk, CHUNK), jnp.float32),
            pltpu.VMEM((nchunk, CHUNK, 2 * TPAD), jnp.float32),
            pltpu.SemaphoreType.DMA,
        ],
        compiler_params=pltpu.CompilerParams(use_tc_tiling_on_sc=True, needs_layout_passes=False),
    )
    def k(table_hbm, sent_hbm, out_hbm, idx_v, q_v, rows_v, sem):
        wid = lax.axis_index("s") * NC + lax.axis_index("c")
        base = wid * b_per_w
        for j in range(nchunk):
            pltpu.sync_copy(sent_hbm.at[pl.ds(base + j * CHUNK, CHUNK)],
                            idx_v.at[j])
        for j in range(nchunk):
            for v in range(nvec):
                sl = pl.ds(v * 16, 16)
                s_vec = idx_v[j, sl]
                q_v[j, sl] = plsc.bitcast(s_vec >> 18, jnp.float32)
                idx_v[j, sl] = s_vec & (QUART - 1)
        copies = [
            pltpu.async_copy(table_hbm.at[idx_v.at[j]], rows_v.at[j], sem)
            for j in range(nchunk)
        ]
        lanes = jnp.full((16,), QLANE, jnp.int32)
        for j in range(nchunk):
            copies[j].wait()
            jj = jnp.full((16,), j, jnp.int32)
            for v in range(nvec):
                rowv = lax.iota(jnp.int32, 16) + (v * 16)
                plsc.store_scatter(rows_v, [jj, rowv, lanes],
                                   q_v[j, pl.ds(v * 16, 16)])
            pltpu.sync_copy(rows_v.at[j],
                            out_hbm.at[pl.ds(base + j * CHUNK, CHUNK)])

    return k(table, sent)


def _final_body(e_ref, o_ref):
    u = lax.bitcast_convert_type(e_ref[...], jnp.uint32)
    q = u[:, QLANE:QLANE + 1]
    sel = jnp.where((q & 2) == 0, u << 16, u & jnp.uint32(0xFFFF0000))
    t128 = lax.bitcast_convert_type(sel, jnp.float32)
    t = jnp.where((q & 1) == 0, t128[:, :TPAD], t128[:, TPAD:])
    t = t[:, :TAGS]
    x = t - jnp.max(t, axis=1, keepdims=True)
    o_ref[...] = x - jnp.log(jnp.sum(jnp.exp(x), axis=1, keepdims=True))


def _final(rows, interpret=False):
    n = rows.shape[0]
    blk = min(n, 8192)
    return pl.pallas_call(
        _final_body,
        grid=(n // blk,),
        in_specs=[
            pl.BlockSpec((blk, 2 * TPAD), lambda i: (i, 0)),
        ],
        out_specs=pl.BlockSpec((blk, TAGS), lambda i: (i, 0)),
        out_shape=jax.ShapeDtypeStruct((n, TAGS), jnp.float32),
        interpret=interpret,
    )(rows)


def kernel(sentence, emb, W1, b1, W2, b2):
    s32 = sentence.astype(jnp.int32)
    wc, bc = _fold(W1, b1, W2, b2)
    pq = _vocab_tags(emb.T, wc, bc)
    rows = _sc_gather(pq, s32)
    return _final(rows)
